# baseline (speedup 1.0000x reference)
import functools

import jax
import jax.numpy as jnp
from jax import lax
from jax.experimental import pallas as pl
from jax.experimental.pallas import tpu as pltpu
from jax.experimental.pallas import tpu_sc as plsc

D = 64
ID_DIM = 56
NC = 2
NS = 16
LANES = 16
NW = NC * NS
CH = 256


def _prep_body(emb_ref, wc_ref, bc_ref, wf_ref, bf_ref, t_ref, u_ref):
    emb = emb_ref[...]
    wf_id = wf_ref[:, :ID_DIM]
    wf_c = wf_ref[:, ID_DIM:]
    dot = functools.partial(
        lax.dot_general,
        precision=lax.Precision.HIGHEST,
        preferred_element_type=jnp.float32,
    )
    t = dot(emb, wf_id, (((1,), (1,)), ((), ())))
    v0 = dot(bc_ref[...], wf_c, (((1,), (1,)), ((), ())))
    t_ref[...] = t + v0 + bf_ref[...]
    u_ref[...] = dot(wc_ref[...], wf_c, (((0,), (1,)), ((), ())))


def _prep(emb_table, Wc, bc, Wf, bf):
    vocab = emb_table.shape[0]
    return pl.pallas_call(
        _prep_body,
        out_shape=(
            jax.ShapeDtypeStruct((vocab, D), jnp.float32),
            jax.ShapeDtypeStruct((1, D), jnp.float32),
        ),
    )(emb_table, Wc, bc.reshape(1, -1), Wf, bf.reshape(1, -1))


def _bcast_lane(vec, lane):
    idx = jnp.full((LANES, 1), lane, jnp.int32)
    dnums = lax.GatherDimensionNumbers(
        offset_dims=(), collapsed_slice_dims=(0,), start_index_map=(0,))
    return lax.gather(vec, idx, dnums, (1,),
                      mode=lax.GatherScatterMode.PROMISE_IN_BOUNDS)


BH = 2048


def _sc_body(t_hbm, ids_hbm, cnt_hbm, u_hbm, out_hbm,
             table_v, ids_v0, ids_v1, cnt_v0, cnt_v1, out_v0, out_v1, u_v,
             sin0, sin1, sout0, sout1, *, n_units, b_dim):
    wid = lax.axis_index("s") * NC + lax.axis_index("c")
    pltpu.sync_copy(t_hbm, table_v)
    pltpu.sync_copy(u_hbm, u_v)
    ids_b, cnt_b, out_b = [ids_v0, ids_v1], [cnt_v0, cnt_v1], [out_v0, out_v1]
    sin, sout = [sin0, sin1], [sout0, sout1]
    per_w = n_units // NW
    base_u = wid * per_w

    def decode(ui):
        return ui >> 3, (ui >> 1) & 3, ui & 1

    def issue_in(ui, b):
        l, _, bh = decode(ui)
        pltpu.async_copy(ids_hbm.at[l, pl.ds(bh * BH, BH)], ids_b[b], sin[b])
        pltpu.async_copy(cnt_hbm.at[l, pl.ds(bh * BH, BH)], cnt_b[b], sin[b])

    def wait_in(b):
        pltpu.make_async_copy(ids_hbm.at[0, pl.ds(0, BH)], ids_b[b],
                              sin[b]).wait()
        pltpu.make_async_copy(cnt_hbm.at[0, pl.ds(0, BH)], cnt_b[b],
                              sin[b]).wait()

    def wait_out(b):
        pltpu.make_async_copy(out_b[b],
                              out_hbm.at[0, pl.ds(0, LANES), pl.ds(0, BH)],
                              sout[b]).wait()

    issue_in(base_u + 0, 0)
    issue_in(base_u + 1, 1)
    n2 = per_w // 2

    def outer(cj, carry):
        for b in range(2):
            ui = base_u + 2 * cj + b
            l, dblk, bh = decode(ui)
            ids_v, cnt_v, out_v = ids_b[b], cnt_b[b], out_b[b]
            wait_in(b)

            @pl.when(cj >= 1)
            def _():
                wait_out(b)

            ublk = u_v[pl.ds(dblk * LANES, LANES)]
            u_splats = [_bcast_lane(ublk, dd) for dd in range(LANES)]
            dbase = dblk * LANES

            @plsc.parallel_loop(0, BH // LANES, unroll=1)
            def group_body(g):
                b16 = g * LANES
                idv = ids_v[pl.ds(b16, LANES)] * D + dbase
                cv = cnt_v[pl.ds(b16, LANES)]
                for dd in range(LANES):
                    val = plsc.load_gather(table_v, [idv + dd])
                    out_v[dd, pl.ds(b16, LANES)] = val + cv * u_splats[dd]

            pltpu.async_copy(
                out_v,
                out_hbm.at[l, pl.ds(dbase, LANES), pl.ds(bh * BH, BH)],
                sout[b])

            @pl.when(cj < n2 - 1)
            def _():
                issue_in(ui + 2, b)
        return carry

    lax.fori_loop(0, n2, outer, 0)
    wait_out(0)
    wait_out(1)


def _sc_lookup(t_flat, ids_t, cnt_t, u_flat):
    l_dim, b_dim = ids_t.shape
    n_units = l_dim * (D // LANES) * (b_dim // BH)
    assert n_units % NW == 0 and (n_units // NW) % 2 == 0
    vocab_d = t_flat.shape[0]
    mesh = plsc.VectorSubcoreMesh(core_axis_name="c", subcore_axis_name="s",
                                  num_cores=NC, num_subcores=NS)
    return pl.kernel(
        functools.partial(_sc_body, n_units=n_units, b_dim=b_dim),
        out_type=jax.ShapeDtypeStruct((l_dim, D, b_dim), jnp.float32),
        mesh=mesh,
        compiler_params=pltpu.CompilerParams(needs_layout_passes=False),
        scratch_types=[
            pltpu.VMEM((vocab_d,), jnp.float32),
            pltpu.VMEM((BH,), jnp.int32),
            pltpu.VMEM((BH,), jnp.int32),
            pltpu.VMEM((BH,), jnp.float32),
            pltpu.VMEM((BH,), jnp.float32),
            pltpu.VMEM((LANES, BH), jnp.float32),
            pltpu.VMEM((LANES, BH), jnp.float32),
            pltpu.VMEM((D,), jnp.float32),
            pltpu.SemaphoreType.DMA,
            pltpu.SemaphoreType.DMA,
            pltpu.SemaphoreType.DMA,
            pltpu.SemaphoreType.DMA,
        ],
    )(t_flat, ids_t, cnt_t, u_flat)


def kernel(relic_ids, counters, emb_table, Wc, bc, Wf, bf):
    b, l = relic_ids.shape
    t, u = _prep(emb_table, Wc, bc, Wf, bf)
    out_t = _sc_lookup(
        t.reshape(-1),
        relic_ids.T.astype(jnp.int32),
        counters.T.astype(jnp.float32),
        u.reshape(-1),
    )
    return jnp.transpose(out_t, (2, 0, 1))


# speedup vs baseline: 1.7014x; 1.7014x over previous
import functools

import jax
import jax.numpy as jnp
from jax import lax
from jax.experimental import pallas as pl
from jax.experimental.pallas import tpu as pltpu
from jax.experimental.pallas import tpu_sc as plsc

D = 64
ID_DIM = 56
NC = 2
NS = 16
LANES = 16
NW = NC * NS
CH = 256


def _prep_body(emb_ref, wc_ref, bc_ref, wf_ref, bf_ref, t_ref, u_ref):
    emb = emb_ref[...]
    wf_id = wf_ref[:, :ID_DIM]
    wf_c = wf_ref[:, ID_DIM:]
    dot = functools.partial(
        lax.dot_general,
        precision=lax.Precision.HIGHEST,
        preferred_element_type=jnp.float32,
    )
    t = dot(emb, wf_id, (((1,), (1,)), ((), ())))
    v0 = dot(bc_ref[...], wf_c, (((1,), (1,)), ((), ())))
    t_ref[...] = t + v0 + bf_ref[...]
    u_ref[...] = dot(wc_ref[...], wf_c, (((0,), (1,)), ((), ())))


def _prep(emb_table, Wc, bc, Wf, bf):
    vocab = emb_table.shape[0]
    return pl.pallas_call(
        _prep_body,
        out_shape=(
            jax.ShapeDtypeStruct((vocab, D), jnp.float32),
            jax.ShapeDtypeStruct((1, D), jnp.float32),
        ),
    )(emb_table, Wc, bc.reshape(1, -1), Wf, bf.reshape(1, -1))


def _bcast_lane(vec, lane):
    idx = jnp.full((LANES, 1), lane, jnp.int32)
    dnums = lax.GatherDimensionNumbers(
        offset_dims=(), collapsed_slice_dims=(0,), start_index_map=(0,))
    return lax.gather(vec, idx, dnums, (1,),
                      mode=lax.GatherScatterMode.PROMISE_IN_BOUNDS)


def _sc_body(t_hbm, ids_hbm, cnt_hbm, u_hbm, out_hbm,
             table_v, ids_v0, ids_v1, cnt_v0, cnt_v1, out_v0, out_v1, u_v,
             sin0, sin1, sout0, sout1, *, n_chunks):
    wid = lax.axis_index("s") * NC + lax.axis_index("c")
    pltpu.sync_copy(t_hbm, table_v)
    pltpu.sync_copy(u_hbm, u_v)
    u_regs = [u_v[pl.ds(LANES * j, LANES)] for j in range(D // LANES)]
    iota = lax.iota(jnp.int32, LANES)
    offs = [iota + LANES * j for j in range(D // LANES)]
    base_w = wid * (n_chunks * CH)
    ids_b, cnt_b, out_b = [ids_v0, ids_v1], [cnt_v0, cnt_v1], [out_v0, out_v1]
    sin, sout = [sin0, sin1], [sout0, sout1]

    def issue_in(ci, b):
        start = base_w + ci * CH
        pltpu.async_copy(ids_hbm.at[pl.ds(start, CH)], ids_b[b], sin[b])
        pltpu.async_copy(cnt_hbm.at[pl.ds(start, CH)], cnt_b[b], sin[b])

    def wait_in(b):
        pltpu.make_async_copy(ids_hbm.at[pl.ds(0, CH)], ids_b[b], sin[b]).wait()
        pltpu.make_async_copy(cnt_hbm.at[pl.ds(0, CH)], cnt_b[b], sin[b]).wait()

    def wait_out(b):
        pltpu.make_async_copy(out_b[b], out_hbm.at[pl.ds(0, CH)], sout[b]).wait()

    issue_in(0, 0)
    issue_in(1, 1)
    n2 = n_chunks // 2

    def outer(cj, carry):
        for b in range(2):
            ci = 2 * cj + b
            ids_v, cnt_v, out_v = ids_b[b], cnt_b[b], out_b[b]
            wait_in(b)

            @pl.when(cj >= 1)
            def _():
                wait_out(b)

            @plsc.parallel_loop(0, CH // LANES, unroll=1)
            def group_body(g):
                b16 = g * LANES
                idv = ids_v[pl.ds(b16, LANES)] * D
                cv = cnt_v[pl.ds(b16, LANES)]
                for e in range(LANES):
                    ide = _bcast_lane(idv, e)
                    ce = _bcast_lane(cv, e)
                    row = b16 + e
                    for j in range(D // LANES):
                        val = plsc.load_gather(table_v, [ide + offs[j]])
                        out_v[row, pl.ds(LANES * j, LANES)] = val + ce * u_regs[j]

            start = base_w + ci * CH
            pltpu.async_copy(out_v, out_hbm.at[pl.ds(start, CH)], sout[b])

            @pl.when(cj < n2 - 1)
            def _():
                issue_in(ci + 2, b)
        return carry

    lax.fori_loop(0, n2, outer, 0)
    wait_out(0)
    wait_out(1)


def _sc_lookup(t_flat, ids_flat, cnt_flat, u_flat):
    n = ids_flat.shape[0]
    assert n % (NW * CH) == 0
    n_chunks = n // (NW * CH)
    vocab_d = t_flat.shape[0]
    mesh = plsc.VectorSubcoreMesh(core_axis_name="c", subcore_axis_name="s",
                                  num_cores=NC, num_subcores=NS)
    return pl.kernel(
        functools.partial(_sc_body, n_chunks=n_chunks),
        out_type=jax.ShapeDtypeStruct((n, D), jnp.float32),
        mesh=mesh,
        compiler_params=pltpu.CompilerParams(needs_layout_passes=False),
        scratch_types=[
            pltpu.VMEM((vocab_d,), jnp.float32),
            pltpu.VMEM((CH,), jnp.int32),
            pltpu.VMEM((CH,), jnp.int32),
            pltpu.VMEM((CH,), jnp.float32),
            pltpu.VMEM((CH,), jnp.float32),
            pltpu.VMEM((CH, D), jnp.float32),
            pltpu.VMEM((CH, D), jnp.float32),
            pltpu.VMEM((D,), jnp.float32),
            pltpu.SemaphoreType.DMA,
            pltpu.SemaphoreType.DMA,
            pltpu.SemaphoreType.DMA,
            pltpu.SemaphoreType.DMA,
        ],
    )(t_flat, ids_flat, cnt_flat, u_flat)


def kernel(relic_ids, counters, emb_table, Wc, bc, Wf, bf):
    b, l = relic_ids.shape
    t, u = _prep(emb_table, Wc, bc, Wf, bf)
    out2d = _sc_lookup(
        t.reshape(-1),
        relic_ids.reshape(-1).astype(jnp.int32),
        counters.reshape(-1).astype(jnp.float32),
        u.reshape(-1),
    )
    return out2d.reshape(b, l, D)
